# Initial kernel scaffold; baseline (speedup 1.0000x reference)
import jax
import jax.numpy as jnp
from jax.experimental import pallas as pl


def kernel(x, edge_index, batch, W1, b1, W2, b2, Wl, bl):
    raise NotImplementedError("write your pallas kernel here")


# speedup vs baseline: 9.1213x; 9.1213x over previous
import functools

import jax
import jax.numpy as jnp
from jax import lax
from jax.experimental import pallas as pl
from jax.experimental.pallas import tpu as pltpu
from jax.experimental.pallas import tpu_sc as plsc

_N = 10000
_NPAD = 10240
_E = 320000
_CHUNK = 128
_NCH = 2560
_EPAD = _NCH * _CHUNK
_NW = 32
_CPW = _NCH // _NW
_NT = 16
_RPT = _NPAD // _NT
_DEGW = 128
_G = 16
_D = 128
_R = 1024
_GRID = _NPAD // _R


def _sc_mesh():
    return plsc.VectorSubcoreMesh(core_axis_name="c", subcore_axis_name="s")


@functools.partial(
    pl.kernel,
    out_type=jax.ShapeDtypeStruct((2, _NPAD, _DEGW), jnp.float32),
    mesh=_sc_mesh(),
    scratch_types=[
        pltpu.VMEM((_CPW, _CHUNK), jnp.int32),
        pltpu.VMEM((_CHUNK, _DEGW), jnp.float32),
        pltpu.VMEM_SHARED((_NPAD, _DEGW), jnp.float32),
    ],
)
def _deg_kernel(dst_hbm, out_hbm, didx_v, rows_v, acc_sh):
    c = lax.axis_index("c")
    s = lax.axis_index("s")
    wid = s * 2 + c

    zero16 = jnp.zeros((16,), jnp.float32)

    def _fillz(i, carry):
        for k in range(_DEGW // 16):
            rows_v[i, pl.ds(k * 16, 16)] = zero16
        return carry

    lax.fori_loop(0, _CHUNK, _fillz, 0)

    pltpu.sync_copy(dst_hbm.at[pl.ds(wid * _CPW, _CPW)], didx_v)

    r0 = s * _RPT
    for i in range(_RPT // _CHUNK):
        pltpu.sync_copy(rows_v, acc_sh.at[pl.ds(r0 + i * _CHUNK, _CHUNK)])
    plsc.subcore_barrier()

    one16 = jnp.full((16,), 1.0, jnp.float32)

    def _fillo(i, carry):
        for k in range(_DEGW // 16):
            rows_v[i, pl.ds(k * 16, 16)] = one16
        return carry

    lax.fori_loop(0, _CHUNK, _fillo, 0)

    def _step(j, carry):
        pltpu.sync_copy(rows_v, acc_sh.at[didx_v.at[j]], add=True)
        return carry

    lax.fori_loop(0, _CPW, _step, 0)
    plsc.subcore_barrier()

    pltpu.sync_copy(acc_sh.at[pl.ds(r0, _RPT)], out_hbm.at[c].at[pl.ds(r0, _RPT)])


@functools.partial(
    pl.kernel,
    out_type=jax.ShapeDtypeStruct((2, _NPAD, _D), jnp.float32),
    mesh=_sc_mesh(),
    scratch_types=[
        pltpu.VMEM((_CPW, _CHUNK), jnp.int32),
        pltpu.VMEM((_CPW, _CHUNK), jnp.int32),
        pltpu.VMEM((_CHUNK, _D), jnp.float32),
        pltpu.VMEM_SHARED((_NPAD, _D), jnp.float32),
        pltpu.SemaphoreType.DMA,
    ],
)
def _msg_kernel(hs_hbm, src_hbm, dst_hbm, out_hbm, sidx_v, didx_v, rows_v,
                acc_sh, sem):
    c = lax.axis_index("c")
    s = lax.axis_index("s")
    wid = s * 2 + c

    zero16 = jnp.zeros((16,), jnp.float32)

    def _fill(i, carry):
        for k in range(_D // 16):
            rows_v[i, pl.ds(k * 16, 16)] = zero16
        return carry

    lax.fori_loop(0, _CHUNK, _fill, 0)

    pltpu.sync_copy(src_hbm.at[pl.ds(wid * _CPW, _CPW)], sidx_v)
    pltpu.sync_copy(dst_hbm.at[pl.ds(wid * _CPW, _CPW)], didx_v)

    r0 = s * _RPT
    for i in range(_RPT // _CHUNK):
        pltpu.sync_copy(rows_v, acc_sh.at[pl.ds(r0 + i * _CHUNK, _CHUNK)])
    plsc.subcore_barrier()

    def _step(j, carry):
        pltpu.async_copy(hs_hbm.at[sidx_v.at[j]], rows_v, sem).wait()
        pltpu.sync_copy(rows_v, acc_sh.at[didx_v.at[j]], add=True)
        return carry

    lax.fori_loop(0, _CPW, _step, 0)
    plsc.subcore_barrier()

    pltpu.sync_copy(acc_sh.at[pl.ds(r0, _RPT)], out_hbm.at[c].at[pl.ds(r0, _RPT)])


def _dinv_block(degp):
    deg = degp[0, :, 0] + degp[1, :, 0] + 1.0
    return lax.rsqrt(deg)


def _tc1_body(x_ref, w1_ref, degp_ref, hs_ref):
    dinv = _dinv_block(degp_ref[...])
    h = jnp.dot(x_ref[...], w1_ref[...], preferred_element_type=jnp.float32)
    hs_ref[...] = h * dinv[:, None]


_tc1 = pl.pallas_call(
    _tc1_body,
    grid=(_GRID,),
    in_specs=[
        pl.BlockSpec((_R, 128), lambda i: (i, 0)),
        pl.BlockSpec((128, 128), lambda i: (0, 0)),
        pl.BlockSpec((2, _R, _DEGW), lambda i: (0, i, 0)),
    ],
    out_specs=pl.BlockSpec((_R, 128), lambda i: (i, 0)),
    out_shape=jax.ShapeDtypeStruct((_NPAD, 128), jnp.float32),
)


def _tc2_body(hs1_ref, p_ref, degp_ref, b1_ref, w2_ref, hs2_ref):
    dinv = _dinv_block(degp_ref[...])
    agg = (p_ref[0] + p_ref[1] + hs1_ref[...]) * dinv[:, None]
    out1 = jnp.maximum(agg + b1_ref[...], 0.0)
    h2 = jnp.dot(out1, w2_ref[...], preferred_element_type=jnp.float32)
    hs2_ref[...] = h2 * dinv[:, None]


_tc2 = pl.pallas_call(
    _tc2_body,
    grid=(_GRID,),
    in_specs=[
        pl.BlockSpec((_R, 128), lambda i: (i, 0)),
        pl.BlockSpec((2, _R, 128), lambda i: (0, i, 0)),
        pl.BlockSpec((2, _R, _DEGW), lambda i: (0, i, 0)),
        pl.BlockSpec((1, 128), lambda i: (0, 0)),
        pl.BlockSpec((128, 128), lambda i: (0, 0)),
    ],
    out_specs=pl.BlockSpec((_R, 128), lambda i: (i, 0)),
    out_shape=jax.ShapeDtypeStruct((_NPAD, 128), jnp.float32),
)


def _tc3_body(hs2_ref, p_ref, degp_ref, b2_ref, batch_ref, wl_ref, bl_ref,
              out_ref, sums, cnts):
    i = pl.program_id(0)
    dinv = _dinv_block(degp_ref[...])
    agg = (p_ref[0] + p_ref[1] + hs2_ref[...]) * dinv[:, None]
    out2 = jnp.maximum(agg + b2_ref[...], 0.0)

    seg = lax.broadcasted_iota(jnp.int32, (_G, _R), 0).astype(jnp.float32)
    m = (seg == batch_ref[...]).astype(jnp.float32)

    @pl.when(i == 0)
    def _():
        sums[...] = jnp.zeros_like(sums)
        cnts[...] = jnp.zeros_like(cnts)

    sums[...] += jnp.dot(m, out2, preferred_element_type=jnp.float32)
    cnts[...] += jnp.sum(m, axis=1, keepdims=True)

    @pl.when(i == _GRID - 1)
    def _():
        pooled = sums[...] / jnp.maximum(cnts[...][:, :1], 1.0)
        out_ref[...] = (jnp.dot(pooled, wl_ref[...],
                                preferred_element_type=jnp.float32)
                        + bl_ref[...])


_tc3 = pl.pallas_call(
    _tc3_body,
    grid=(_GRID,),
    in_specs=[
        pl.BlockSpec((_R, 128), lambda i: (i, 0)),
        pl.BlockSpec((2, _R, 128), lambda i: (0, i, 0)),
        pl.BlockSpec((2, _R, _DEGW), lambda i: (0, i, 0)),
        pl.BlockSpec((1, 128), lambda i: (0, 0)),
        pl.BlockSpec((1, _R), lambda i: (0, i)),
        pl.BlockSpec((128, _G), lambda i: (0, 0)),
        pl.BlockSpec((1, _G), lambda i: (0, 0)),
    ],
    out_specs=pl.BlockSpec((_G, _G), lambda i: (0, 0)),
    out_shape=jax.ShapeDtypeStruct((_G, _G), jnp.float32),
    scratch_shapes=[
        pltpu.VMEM((_G, 128), jnp.float32),
        pltpu.VMEM((_G, 128), jnp.float32),
    ],
)


def kernel(x, edge_index, batch, W1, b1, W2, b2, Wl, bl):
    f32 = jnp.float32
    xp = jnp.concatenate([x, jnp.zeros((_NPAD - _N, x.shape[1]), x.dtype)],
                         axis=0)
    pad_e = _EPAD - _E
    src = jnp.concatenate(
        [edge_index[0], jnp.zeros((pad_e,), jnp.int32)]).reshape(_NCH, _CHUNK)
    dst = jnp.concatenate(
        [edge_index[1],
         jnp.full((pad_e,), _NPAD - 1, jnp.int32)]).reshape(_NCH, _CHUNK)
    batchp = jnp.concatenate(
        [batch, jnp.full((_NPAD - _N,), 255, batch.dtype)]
    ).astype(f32).reshape(1, _NPAD)

    w2p = jnp.zeros((128, 128), f32).at[:, :64].set(W2)
    b2p = jnp.zeros((1, 128), f32).at[0, :64].set(b2)
    wlp = jnp.zeros((128, _G), f32).at[:64, :].set(Wl)

    degp = _deg_kernel(dst)
    hs1 = _tc1(xp, W1, degp)
    p1 = _msg_kernel(hs1, src, dst)
    hs2 = _tc2(hs1, p1, degp, b1.reshape(1, -1), w2p)
    p2 = _msg_kernel(hs2, src, dst)
    out = _tc3(hs2, p2, degp, b2p, batchp, wlp, bl.reshape(1, -1))
    return out


# speedup vs baseline: 28.1234x; 3.0833x over previous
import functools

import jax
import jax.numpy as jnp
from jax import lax
from jax.experimental import pallas as pl
from jax.experimental.pallas import tpu as pltpu
from jax.experimental.pallas import tpu_sc as plsc

_N = 10000
_NPAD = 10240
_E = 320000
_CHUNK = 128
_NCH = 2560
_EPAD = _NCH * _CHUNK
_NW = 32
_CPW = _NCH // _NW
_NT = 16
_RPT = _NPAD // _NT
_DEGW = 128
_G = 16
_D = 128
_R = 1024
_GRID = _NPAD // _R


def _sc_mesh():
    return plsc.VectorSubcoreMesh(core_axis_name="c", subcore_axis_name="s")


@functools.partial(
    pl.kernel,
    out_type=jax.ShapeDtypeStruct((2, _NPAD, _DEGW), jnp.float32),
    mesh=_sc_mesh(),
    scratch_types=[
        pltpu.VMEM((_CPW, _CHUNK), jnp.int32),
        pltpu.VMEM((_CHUNK, _DEGW), jnp.float32),
        pltpu.VMEM_SHARED((_NPAD, _DEGW), jnp.float32),
        pltpu.SemaphoreType.DMA,
    ],
)
def _deg_kernel(dst_hbm, out_hbm, didx_v, rows_v, acc_sh, sem):
    c = lax.axis_index("c")
    s = lax.axis_index("s")
    wid = s * 2 + c

    zero16 = jnp.zeros((16,), jnp.float32)

    def _fillz(i, carry):
        for k in range(_DEGW // 16):
            rows_v[i, pl.ds(k * 16, 16)] = zero16
        return carry

    lax.fori_loop(0, _CHUNK, _fillz, 0)

    pltpu.sync_copy(dst_hbm.at[pl.ds(wid * _CPW, _CPW)], didx_v)

    r0 = s * _RPT
    for i in range(_RPT // _CHUNK):
        pltpu.sync_copy(rows_v, acc_sh.at[pl.ds(r0 + i * _CHUNK, _CHUNK)])
    plsc.subcore_barrier()

    one16 = jnp.full((16,), 1.0, jnp.float32)

    def _fillo(i, carry):
        for k in range(_DEGW // 16):
            rows_v[i, pl.ds(k * 16, 16)] = one16
        return carry

    lax.fori_loop(0, _CHUNK, _fillo, 0)

    def _step(j, carry):
        pltpu.async_copy(rows_v, acc_sh.at[didx_v.at[j]], sem, add=True)
        return carry

    lax.fori_loop(0, _CPW, _step, 0)

    def _drain(j, carry):
        pltpu.make_async_copy(rows_v, acc_sh.at[didx_v.at[0]], sem).wait()
        return carry

    lax.fori_loop(0, _CPW, _drain, 0)
    plsc.subcore_barrier()

    pltpu.sync_copy(acc_sh.at[pl.ds(r0, _RPT)], out_hbm.at[c].at[pl.ds(r0, _RPT)])


_NBUF = 2
_GRP = 16
_NGRP = _CPW // _GRP


@functools.partial(
    pl.kernel,
    out_type=jax.ShapeDtypeStruct((2, _NPAD, _D), jnp.float32),
    mesh=_sc_mesh(),
    scratch_types=[
        pltpu.VMEM((2, _GRP, _CHUNK), jnp.int32),
        pltpu.VMEM((2, _GRP, _CHUNK), jnp.int32),
        pltpu.VMEM((_CHUNK, _D), jnp.float32),
        pltpu.VMEM((_CHUNK, _D), jnp.float32),
        pltpu.VMEM_SHARED((_NPAD, _D), jnp.float32),
        pltpu.SemaphoreType.DMA,
        pltpu.SemaphoreType.DMA,
    ],
)
def _msg_kernel(hs_hbm, src_hbm, dst_hbm, out_hbm, sidx_v, didx_v,
                r0v, r1v, acc_sh, sem0, sem1):
    c = lax.axis_index("c")
    s = lax.axis_index("s")
    wid = s * 2 + c
    bufs = (r0v, r1v)
    sems = (sem0, sem1)

    zero16 = jnp.zeros((16,), jnp.float32)

    def _fill(i, carry):
        for k in range(_D // 16):
            r0v[i, pl.ds(k * 16, 16)] = zero16
        return carry

    lax.fori_loop(0, _CHUNK, _fill, 0)

    r0 = s * _RPT
    for i in range(_RPT // _CHUNK):
        pltpu.sync_copy(r0v, acc_sh.at[pl.ds(r0 + i * _CHUNK, _CHUNK)])
    plsc.subcore_barrier()

    base = wid * _CPW
    pltpu.sync_copy(src_hbm.at[pl.ds(base, _GRP)], sidx_v.at[0])
    pltpu.sync_copy(dst_hbm.at[pl.ds(base, _GRP)], didx_v.at[0])

    for b in range(_NBUF):
        pltpu.async_copy(hs_hbm.at[sidx_v.at[0, b]], bufs[b], sems[b])

    def _grp(g, carry):
        slot = lax.rem(g, 2)
        nslot = lax.rem(g + 1, 2)

        @pl.when(g + 1 < _NGRP)
        def _():
            pltpu.sync_copy(src_hbm.at[pl.ds(base + (g + 1) * _GRP, _GRP)],
                            sidx_v.at[nslot])
            pltpu.sync_copy(dst_hbm.at[pl.ds(base + (g + 1) * _GRP, _GRP)],
                            didx_v.at[nslot])

        def _pair(t, carry2):
            for b in range(_NBUF):
                jl = 2 * t + b
                pltpu.make_async_copy(hs_hbm.at[sidx_v.at[0, 0]], bufs[b],
                                      sems[b]).wait()
                pltpu.sync_copy(bufs[b], acc_sh.at[didx_v.at[slot, jl]],
                                add=True)
                pltpu.async_copy(hs_hbm.at[sidx_v.at[slot, jl + 2]],
                                 bufs[b], sems[b])
            return carry2

        lax.fori_loop(0, _GRP // 2 - 1, _pair, 0)

        for b in range(_NBUF):
            jl = _GRP - 2 + b
            pltpu.make_async_copy(hs_hbm.at[sidx_v.at[0, 0]], bufs[b],
                                  sems[b]).wait()
            pltpu.sync_copy(bufs[b], acc_sh.at[didx_v.at[slot, jl]],
                            add=True)

            @pl.when(g + 1 < _NGRP)
            def _():
                pltpu.async_copy(hs_hbm.at[sidx_v.at[nslot, b]],
                                 bufs[b], sems[b])
        return carry

    lax.fori_loop(0, _NGRP, _grp, 0)
    plsc.subcore_barrier()

    pltpu.sync_copy(acc_sh.at[pl.ds(r0, _RPT)], out_hbm.at[c].at[pl.ds(r0, _RPT)])


def _dinv_block(degp):
    deg = degp[0, :, 0] + degp[1, :, 0] + 1.0
    return lax.rsqrt(deg)


def _tc1_body(x_ref, w1_ref, degp_ref, hs_ref):
    dinv = _dinv_block(degp_ref[...])
    h = jnp.dot(x_ref[...], w1_ref[...], preferred_element_type=jnp.float32)
    hs_ref[...] = h * dinv[:, None]


_tc1 = pl.pallas_call(
    _tc1_body,
    grid=(_GRID,),
    in_specs=[
        pl.BlockSpec((_R, 128), lambda i: (i, 0)),
        pl.BlockSpec((128, 128), lambda i: (0, 0)),
        pl.BlockSpec((2, _R, _DEGW), lambda i: (0, i, 0)),
    ],
    out_specs=pl.BlockSpec((_R, 128), lambda i: (i, 0)),
    out_shape=jax.ShapeDtypeStruct((_NPAD, 128), jnp.float32),
)


def _tc2_body(hs1_ref, p_ref, degp_ref, b1_ref, w2_ref, hs2_ref):
    dinv = _dinv_block(degp_ref[...])
    agg = (p_ref[0] + p_ref[1] + hs1_ref[...]) * dinv[:, None]
    out1 = jnp.maximum(agg + b1_ref[...], 0.0)
    h2 = jnp.dot(out1, w2_ref[...], preferred_element_type=jnp.float32)
    hs2_ref[...] = h2 * dinv[:, None]


_tc2 = pl.pallas_call(
    _tc2_body,
    grid=(_GRID,),
    in_specs=[
        pl.BlockSpec((_R, 128), lambda i: (i, 0)),
        pl.BlockSpec((2, _R, 128), lambda i: (0, i, 0)),
        pl.BlockSpec((2, _R, _DEGW), lambda i: (0, i, 0)),
        pl.BlockSpec((1, 128), lambda i: (0, 0)),
        pl.BlockSpec((128, 128), lambda i: (0, 0)),
    ],
    out_specs=pl.BlockSpec((_R, 128), lambda i: (i, 0)),
    out_shape=jax.ShapeDtypeStruct((_NPAD, 128), jnp.float32),
)


def _tc3_body(hs2_ref, p_ref, degp_ref, b2_ref, batch_ref, wl_ref, bl_ref,
              out_ref, sums, cnts):
    i = pl.program_id(0)
    dinv = _dinv_block(degp_ref[...])
    agg = (p_ref[0] + p_ref[1] + hs2_ref[...]) * dinv[:, None]
    out2 = jnp.maximum(agg + b2_ref[...], 0.0)

    seg = lax.broadcasted_iota(jnp.int32, (_G, _R), 0).astype(jnp.float32)
    m = (seg == batch_ref[...]).astype(jnp.float32)

    @pl.when(i == 0)
    def _():
        sums[...] = jnp.zeros_like(sums)
        cnts[...] = jnp.zeros_like(cnts)

    sums[...] += jnp.dot(m, out2, preferred_element_type=jnp.float32)
    cnts[...] += jnp.sum(m, axis=1, keepdims=True)

    @pl.when(i == _GRID - 1)
    def _():
        pooled = sums[...] / jnp.maximum(cnts[...][:, :1], 1.0)
        out_ref[...] = (jnp.dot(pooled, wl_ref[...],
                                preferred_element_type=jnp.float32)
                        + bl_ref[...])


_tc3 = pl.pallas_call(
    _tc3_body,
    grid=(_GRID,),
    in_specs=[
        pl.BlockSpec((_R, 128), lambda i: (i, 0)),
        pl.BlockSpec((2, _R, 128), lambda i: (0, i, 0)),
        pl.BlockSpec((2, _R, _DEGW), lambda i: (0, i, 0)),
        pl.BlockSpec((1, 128), lambda i: (0, 0)),
        pl.BlockSpec((1, _R), lambda i: (0, i)),
        pl.BlockSpec((128, _G), lambda i: (0, 0)),
        pl.BlockSpec((1, _G), lambda i: (0, 0)),
    ],
    out_specs=pl.BlockSpec((_G, _G), lambda i: (0, 0)),
    out_shape=jax.ShapeDtypeStruct((_G, _G), jnp.float32),
    scratch_shapes=[
        pltpu.VMEM((_G, 128), jnp.float32),
        pltpu.VMEM((_G, 128), jnp.float32),
    ],
)


def kernel(x, edge_index, batch, W1, b1, W2, b2, Wl, bl):
    f32 = jnp.float32
    xp = jnp.concatenate([x, jnp.zeros((_NPAD - _N, x.shape[1]), x.dtype)],
                         axis=0)
    pad_e = _EPAD - _E
    pad_iota = jnp.arange(pad_e, dtype=jnp.int32)
    src = jnp.concatenate(
        [edge_index[0], pad_iota % _N]).reshape(_NCH, _CHUNK)
    dst = jnp.concatenate(
        [edge_index[1], _N + pad_iota % (_NPAD - _N)]).reshape(_NCH, _CHUNK)
    batchp = jnp.concatenate(
        [batch, jnp.full((_NPAD - _N,), 255, batch.dtype)]
    ).astype(f32).reshape(1, _NPAD)

    w2p = jnp.zeros((128, 128), f32).at[:, :64].set(W2)
    b2p = jnp.zeros((1, 128), f32).at[0, :64].set(b2)
    wlp = jnp.zeros((128, _G), f32).at[:64, :].set(Wl)

    degp = _deg_kernel(dst)
    hs1 = _tc1(xp, W1, degp)
    p1 = _msg_kernel(hs1, src, dst)
    hs2 = _tc2(hs1, p1, degp, b1.reshape(1, -1), w2p)
    p2 = _msg_kernel(hs2, src, dst)
    out = _tc3(hs2, p2, degp, b2p, batchp, wlp, bl.reshape(1, -1))
    return out


# speedup vs baseline: 30.4294x; 1.0820x over previous
import functools

import jax
import jax.numpy as jnp
from jax import lax
from jax.experimental import pallas as pl
from jax.experimental.pallas import tpu as pltpu
from jax.experimental.pallas import tpu_sc as plsc

_N = 10000
_NPAD = 10240
_E = 320000
_CHUNK = 128
_NCH = 2560
_EPAD = _NCH * _CHUNK
_NW = 32
_CPW = _NCH // _NW
_NT = 16
_RPT = _NPAD // _NT
_DEGW = 64
_G = 16
_D = 128
_R = 1024
_GRID = _NPAD // _R


def _sc_mesh():
    return plsc.VectorSubcoreMesh(core_axis_name="c", subcore_axis_name="s")


@functools.partial(
    pl.kernel,
    out_type=jax.ShapeDtypeStruct((2, _NPAD, _DEGW), jnp.float32),
    mesh=_sc_mesh(),
    scratch_types=[
        pltpu.VMEM((_CPW, _CHUNK), jnp.int32),
        pltpu.VMEM((_CHUNK, _DEGW), jnp.float32),
        pltpu.VMEM_SHARED((_NPAD, _DEGW), jnp.float32),
        pltpu.SemaphoreType.DMA,
    ],
)
def _deg_kernel(dst_hbm, out_hbm, didx_v, rows_v, acc_sh, sem):
    c = lax.axis_index("c")
    s = lax.axis_index("s")
    wid = s * 2 + c

    zero16 = jnp.zeros((16,), jnp.float32)

    def _fillz(i, carry):
        for k in range(_DEGW // 16):
            rows_v[i, pl.ds(k * 16, 16)] = zero16
        return carry

    lax.fori_loop(0, _CHUNK, _fillz, 0)

    pltpu.sync_copy(dst_hbm.at[pl.ds(wid * _CPW, _CPW)], didx_v)

    r0 = s * _RPT
    for i in range(_RPT // _CHUNK):
        pltpu.sync_copy(rows_v, acc_sh.at[pl.ds(r0 + i * _CHUNK, _CHUNK)])
    plsc.subcore_barrier()

    one16 = jnp.full((16,), 1.0, jnp.float32)

    def _fillo(i, carry):
        for k in range(_DEGW // 16):
            rows_v[i, pl.ds(k * 16, 16)] = one16
        return carry

    lax.fori_loop(0, _CHUNK, _fillo, 0)

    def _step(j, carry):
        pltpu.async_copy(rows_v, acc_sh.at[didx_v.at[j]], sem, add=True)
        return carry

    lax.fori_loop(0, _CPW, _step, 0)

    def _drain(j, carry):
        pltpu.make_async_copy(rows_v, acc_sh.at[didx_v.at[0]], sem).wait()
        return carry

    lax.fori_loop(0, _CPW, _drain, 0)
    plsc.subcore_barrier()

    pltpu.sync_copy(acc_sh.at[pl.ds(r0, _RPT)], out_hbm.at[c].at[pl.ds(r0, _RPT)])


_NBUF = 2
_GRP = 16
_NGRP = _CPW // _GRP


@functools.partial(
    pl.kernel,
    out_type=jax.ShapeDtypeStruct((2, _NPAD, _D), jnp.float32),
    mesh=_sc_mesh(),
    scratch_types=[
        pltpu.VMEM((2, _GRP, _CHUNK), jnp.int32),
        pltpu.VMEM((2, _GRP, _CHUNK), jnp.int32),
        pltpu.VMEM((_CHUNK, _D), jnp.float32),
        pltpu.VMEM((_CHUNK, _D), jnp.float32),
        pltpu.VMEM_SHARED((_NPAD, _D), jnp.float32),
        pltpu.SemaphoreType.DMA,
        pltpu.SemaphoreType.DMA,
    ],
)
def _msg_kernel(hs_hbm, src_hbm, dst_hbm, out_hbm, sidx_v, didx_v,
                r0v, r1v, acc_sh, sem0, sem1):
    c = lax.axis_index("c")
    s = lax.axis_index("s")
    wid = s * 2 + c
    bufs = (r0v, r1v)
    sems = (sem0, sem1)

    zero16 = jnp.zeros((16,), jnp.float32)

    def _fill(i, carry):
        for k in range(_D // 16):
            r0v[i, pl.ds(k * 16, 16)] = zero16
        return carry

    lax.fori_loop(0, _CHUNK, _fill, 0)

    r0 = s * _RPT
    for i in range(_RPT // _CHUNK):
        pltpu.sync_copy(r0v, acc_sh.at[pl.ds(r0 + i * _CHUNK, _CHUNK)])
    plsc.subcore_barrier()

    base = wid * _CPW
    pltpu.sync_copy(src_hbm.at[pl.ds(base, _GRP)], sidx_v.at[0])
    pltpu.sync_copy(dst_hbm.at[pl.ds(base, _GRP)], didx_v.at[0])

    for b in range(_NBUF):
        pltpu.async_copy(hs_hbm.at[sidx_v.at[0, b]], bufs[b], sems[b])

    def _grp(g, carry):
        slot = lax.rem(g, 2)
        nslot = lax.rem(g + 1, 2)

        @pl.when(g + 1 < _NGRP)
        def _():
            pltpu.sync_copy(src_hbm.at[pl.ds(base + (g + 1) * _GRP, _GRP)],
                            sidx_v.at[nslot])
            pltpu.sync_copy(dst_hbm.at[pl.ds(base + (g + 1) * _GRP, _GRP)],
                            didx_v.at[nslot])

        def _pair(t, carry2):
            for b in range(_NBUF):
                jl = 2 * t + b
                pltpu.make_async_copy(hs_hbm.at[sidx_v.at[0, 0]], bufs[b],
                                      sems[b]).wait()
                pltpu.sync_copy(bufs[b], acc_sh.at[didx_v.at[slot, jl]],
                                add=True)
                pltpu.async_copy(hs_hbm.at[sidx_v.at[slot, jl + 2]],
                                 bufs[b], sems[b])
            return carry2

        lax.fori_loop(0, _GRP // 2 - 1, _pair, 0)

        for b in range(_NBUF):
            jl = _GRP - 2 + b
            pltpu.make_async_copy(hs_hbm.at[sidx_v.at[0, 0]], bufs[b],
                                  sems[b]).wait()
            pltpu.sync_copy(bufs[b], acc_sh.at[didx_v.at[slot, jl]],
                            add=True)

            @pl.when(g + 1 < _NGRP)
            def _():
                pltpu.async_copy(hs_hbm.at[sidx_v.at[nslot, b]],
                                 bufs[b], sems[b])
        return carry

    lax.fori_loop(0, _NGRP, _grp, 0)
    plsc.subcore_barrier()

    pltpu.sync_copy(acc_sh.at[pl.ds(r0, _RPT)], out_hbm.at[c].at[pl.ds(r0, _RPT)])


def _dinv_block(degp):
    deg = degp[0, :, 0] + degp[1, :, 0] + 1.0
    return lax.rsqrt(deg)


def _tc1_body(x_ref, w1_ref, degp_ref, hs_ref):
    dinv = _dinv_block(degp_ref[...])
    h = jnp.dot(x_ref[...], w1_ref[...], preferred_element_type=jnp.float32)
    hs_ref[...] = h * dinv[:, None]


_tc1 = pl.pallas_call(
    _tc1_body,
    grid=(_GRID,),
    in_specs=[
        pl.BlockSpec((_R, 128), lambda i: (i, 0)),
        pl.BlockSpec((128, 128), lambda i: (0, 0)),
        pl.BlockSpec((2, _R, _DEGW), lambda i: (0, i, 0)),
    ],
    out_specs=pl.BlockSpec((_R, 128), lambda i: (i, 0)),
    out_shape=jax.ShapeDtypeStruct((_NPAD, 128), jnp.float32),
)


def _tc2_body(hs1_ref, p_ref, degp_ref, b1_ref, w2_ref, hs2_ref):
    dinv = _dinv_block(degp_ref[...])
    agg = (p_ref[0] + p_ref[1] + hs1_ref[...]) * dinv[:, None]
    out1 = jnp.maximum(agg + b1_ref[...], 0.0)
    h2 = jnp.dot(out1, w2_ref[...], preferred_element_type=jnp.float32)
    hs2_ref[...] = h2 * dinv[:, None]


_tc2 = pl.pallas_call(
    _tc2_body,
    grid=(_GRID,),
    in_specs=[
        pl.BlockSpec((_R, 128), lambda i: (i, 0)),
        pl.BlockSpec((2, _R, 128), lambda i: (0, i, 0)),
        pl.BlockSpec((2, _R, _DEGW), lambda i: (0, i, 0)),
        pl.BlockSpec((1, 128), lambda i: (0, 0)),
        pl.BlockSpec((128, 128), lambda i: (0, 0)),
    ],
    out_specs=pl.BlockSpec((_R, 128), lambda i: (i, 0)),
    out_shape=jax.ShapeDtypeStruct((_NPAD, 128), jnp.float32),
)


def _tc3_body(hs2_ref, p_ref, degp_ref, b2_ref, batch_ref, wl_ref, bl_ref,
              out_ref, sums, cnts):
    i = pl.program_id(0)
    dinv = _dinv_block(degp_ref[...])
    agg = (p_ref[0] + p_ref[1] + hs2_ref[...]) * dinv[:, None]
    out2 = jnp.maximum(agg + b2_ref[...], 0.0)

    seg = lax.broadcasted_iota(jnp.int32, (_G, _R), 0).astype(jnp.float32)
    m = (seg == batch_ref[...]).astype(jnp.float32)

    @pl.when(i == 0)
    def _():
        sums[...] = jnp.zeros_like(sums)
        cnts[...] = jnp.zeros_like(cnts)

    sums[...] += jnp.dot(m, out2, preferred_element_type=jnp.float32)
    cnts[...] += jnp.sum(m, axis=1, keepdims=True)

    @pl.when(i == _GRID - 1)
    def _():
        pooled = sums[...] / jnp.maximum(cnts[...][:, :1], 1.0)
        out_ref[...] = (jnp.dot(pooled, wl_ref[...],
                                preferred_element_type=jnp.float32)
                        + bl_ref[...])


_tc3 = pl.pallas_call(
    _tc3_body,
    grid=(_GRID,),
    in_specs=[
        pl.BlockSpec((_R, 128), lambda i: (i, 0)),
        pl.BlockSpec((2, _R, 128), lambda i: (0, i, 0)),
        pl.BlockSpec((2, _R, _DEGW), lambda i: (0, i, 0)),
        pl.BlockSpec((1, 128), lambda i: (0, 0)),
        pl.BlockSpec((1, _R), lambda i: (0, i)),
        pl.BlockSpec((128, _G), lambda i: (0, 0)),
        pl.BlockSpec((1, _G), lambda i: (0, 0)),
    ],
    out_specs=pl.BlockSpec((_G, _G), lambda i: (0, 0)),
    out_shape=jax.ShapeDtypeStruct((_G, _G), jnp.float32),
    scratch_shapes=[
        pltpu.VMEM((_G, 128), jnp.float32),
        pltpu.VMEM((_G, 128), jnp.float32),
    ],
)


def kernel(x, edge_index, batch, W1, b1, W2, b2, Wl, bl):
    f32 = jnp.float32
    xp = jnp.concatenate([x, jnp.zeros((_NPAD - _N, x.shape[1]), x.dtype)],
                         axis=0)
    pad_e = _EPAD - _E
    pad_iota = jnp.arange(pad_e, dtype=jnp.int32)
    src = jnp.concatenate(
        [edge_index[0], pad_iota % _N]).reshape(_NCH, _CHUNK)
    dst = jnp.concatenate(
        [edge_index[1], _N + pad_iota % (_NPAD - _N)]).reshape(_NCH, _CHUNK)
    batchp = jnp.concatenate(
        [batch, jnp.full((_NPAD - _N,), 255, batch.dtype)]
    ).astype(f32).reshape(1, _NPAD)

    w2p = jnp.zeros((128, 128), f32).at[:, :64].set(W2)
    b2p = jnp.zeros((1, 128), f32).at[0, :64].set(b2)
    wlp = jnp.zeros((128, _G), f32).at[:64, :].set(Wl)

    degp = _deg_kernel(dst)
    hs1 = _tc1(xp, W1, degp)
    p1 = _msg_kernel(hs1, src, dst)
    hs2 = _tc2(hs1, p1, degp, b1.reshape(1, -1), w2p)
    p2 = _msg_kernel(hs2, src, dst)
    out = _tc3(hs2, p2, degp, b2p, batchp, wlp, bl.reshape(1, -1))
    return out


# speedup vs baseline: 30.4784x; 1.0016x over previous
import functools

import jax
import jax.numpy as jnp
from jax import lax
from jax.experimental import pallas as pl
from jax.experimental.pallas import tpu as pltpu
from jax.experimental.pallas import tpu_sc as plsc

_N = 10000
_NPAD = 10240
_E = 320000
_CHUNK = 128
_NCH = 2560
_EPAD = _NCH * _CHUNK
_NW = 32
_CPW = _NCH // _NW
_NT = 16
_RPT = _NPAD // _NT
_DEGW = 64
_G = 16
_D = 128
_R = 1024
_GRID = _NPAD // _R


def _sc_mesh():
    return plsc.VectorSubcoreMesh(core_axis_name="c", subcore_axis_name="s")


@functools.partial(
    pl.kernel,
    out_type=jax.ShapeDtypeStruct((2, _NPAD, _DEGW), jnp.float32),
    mesh=_sc_mesh(),
    scratch_types=[
        pltpu.VMEM((_CPW, _CHUNK), jnp.int32),
        pltpu.VMEM((_CHUNK, _DEGW), jnp.float32),
        pltpu.VMEM_SHARED((_NPAD, _DEGW), jnp.float32),
        pltpu.SemaphoreType.DMA,
    ],
)
def _deg_kernel(dst_hbm, out_hbm, didx_v, rows_v, acc_sh, sem):
    c = lax.axis_index("c")
    s = lax.axis_index("s")
    wid = s * 2 + c

    zero16 = jnp.zeros((16,), jnp.float32)

    def _fillz(i, carry):
        for k in range(_DEGW // 16):
            rows_v[i, pl.ds(k * 16, 16)] = zero16
        return carry

    lax.fori_loop(0, _CHUNK, _fillz, 0)

    pltpu.sync_copy(dst_hbm.at[pl.ds(wid * _CPW, _CPW)], didx_v)

    r0 = s * _RPT
    for i in range(_RPT // _CHUNK):
        pltpu.sync_copy(rows_v, acc_sh.at[pl.ds(r0 + i * _CHUNK, _CHUNK)])
    plsc.subcore_barrier()

    one16 = jnp.full((16,), 1.0, jnp.float32)

    def _fillo(i, carry):
        for k in range(_DEGW // 16):
            rows_v[i, pl.ds(k * 16, 16)] = one16
        return carry

    lax.fori_loop(0, _CHUNK, _fillo, 0)

    def _step(j, carry):
        pltpu.async_copy(rows_v, acc_sh.at[didx_v.at[j]], sem, add=True)
        return carry

    lax.fori_loop(0, _CPW, _step, 0)

    def _drain(j, carry):
        pltpu.make_async_copy(rows_v, acc_sh.at[didx_v.at[0]], sem).wait()
        return carry

    lax.fori_loop(0, _CPW, _drain, 0)
    plsc.subcore_barrier()

    pltpu.sync_copy(acc_sh.at[pl.ds(r0, _RPT)], out_hbm.at[c].at[pl.ds(r0, _RPT)])


_NBUF = 2
_GRP = 16
_NGRP = _CPW // _GRP


def _make_msg_kernel(width):
    @functools.partial(
        pl.kernel,
        out_type=jax.ShapeDtypeStruct((2, _NPAD, width), jnp.float32),
        mesh=_sc_mesh(),
        scratch_types=[
            pltpu.VMEM((2, _GRP, _CHUNK), jnp.int32),
            pltpu.VMEM((2, _GRP, _CHUNK), jnp.int32),
            pltpu.VMEM((_CHUNK, width), jnp.float32),
            pltpu.VMEM((_CHUNK, width), jnp.float32),
            pltpu.VMEM_SHARED((_NPAD, width), jnp.float32),
            pltpu.SemaphoreType.DMA,
            pltpu.SemaphoreType.DMA,
        ],
    )
    def _msg_kernel(hs_hbm, src_hbm, dst_hbm, out_hbm, sidx_v, didx_v,
                    r0v, r1v, acc_sh, sem0, sem1):
        c = lax.axis_index("c")
        s = lax.axis_index("s")
        wid = s * 2 + c
        bufs = (r0v, r1v)
        sems = (sem0, sem1)

        zero16 = jnp.zeros((16,), jnp.float32)

        def _fill(i, carry):
            for k in range(width // 16):
                r0v[i, pl.ds(k * 16, 16)] = zero16
            return carry

        lax.fori_loop(0, _CHUNK, _fill, 0)

        r0 = s * _RPT
        for i in range(_RPT // _CHUNK):
            pltpu.sync_copy(r0v, acc_sh.at[pl.ds(r0 + i * _CHUNK, _CHUNK)])
        plsc.subcore_barrier()

        base = wid * _CPW
        pltpu.sync_copy(src_hbm.at[pl.ds(base, _GRP)], sidx_v.at[0])
        pltpu.sync_copy(dst_hbm.at[pl.ds(base, _GRP)], didx_v.at[0])

        for b in range(_NBUF):
            pltpu.async_copy(hs_hbm.at[sidx_v.at[0, b]], bufs[b], sems[b])

        def _grp(g, carry):
            slot = lax.rem(g, 2)
            nslot = lax.rem(g + 1, 2)

            @pl.when(g + 1 < _NGRP)
            def _():
                pltpu.sync_copy(src_hbm.at[pl.ds(base + (g + 1) * _GRP, _GRP)],
                                sidx_v.at[nslot])
                pltpu.sync_copy(dst_hbm.at[pl.ds(base + (g + 1) * _GRP, _GRP)],
                                didx_v.at[nslot])

            def _pair(t, carry2):
                for b in range(_NBUF):
                    jl = 2 * t + b
                    pltpu.make_async_copy(hs_hbm.at[sidx_v.at[0, 0]], bufs[b],
                                          sems[b]).wait()
                    pltpu.sync_copy(bufs[b], acc_sh.at[didx_v.at[slot, jl]],
                                    add=True)
                    pltpu.async_copy(hs_hbm.at[sidx_v.at[slot, jl + 2]],
                                     bufs[b], sems[b])
                return carry2

            lax.fori_loop(0, _GRP // 2 - 1, _pair, 0)

            for b in range(_NBUF):
                jl = _GRP - 2 + b
                pltpu.make_async_copy(hs_hbm.at[sidx_v.at[0, 0]], bufs[b],
                                      sems[b]).wait()
                pltpu.sync_copy(bufs[b], acc_sh.at[didx_v.at[slot, jl]],
                                add=True)

                @pl.when(g + 1 < _NGRP)
                def _():
                    pltpu.async_copy(hs_hbm.at[sidx_v.at[nslot, b]],
                                     bufs[b], sems[b])
            return carry

        lax.fori_loop(0, _NGRP, _grp, 0)
        plsc.subcore_barrier()

        pltpu.sync_copy(acc_sh.at[pl.ds(r0, _RPT)],
                        out_hbm.at[c].at[pl.ds(r0, _RPT)])

    return _msg_kernel


_msg_kernel = _make_msg_kernel(_D)


def _dinv_block(degp):
    deg = degp[0, :, 0] + degp[1, :, 0] + 1.0
    return lax.rsqrt(deg)


def _tc1_body(x_ref, w1_ref, degp_ref, hs_ref):
    dinv = _dinv_block(degp_ref[...])
    h = jnp.dot(x_ref[...], w1_ref[...], preferred_element_type=jnp.float32)
    hs_ref[...] = h * dinv[:, None]


_tc1 = pl.pallas_call(
    _tc1_body,
    grid=(_GRID,),
    in_specs=[
        pl.BlockSpec((_R, 128), lambda i: (i, 0)),
        pl.BlockSpec((128, 128), lambda i: (0, 0)),
        pl.BlockSpec((2, _R, _DEGW), lambda i: (0, i, 0)),
    ],
    out_specs=pl.BlockSpec((_R, 128), lambda i: (i, 0)),
    out_shape=jax.ShapeDtypeStruct((_NPAD, 128), jnp.float32),
)


def _tc2_body(hs1_ref, p_ref, degp_ref, b1_ref, w2_ref, hs2_ref):
    dinv = _dinv_block(degp_ref[...])
    agg = (p_ref[0] + p_ref[1] + hs1_ref[...]) * dinv[:, None]
    out1 = jnp.maximum(agg + b1_ref[...], 0.0)
    h2 = jnp.dot(out1, w2_ref[...], preferred_element_type=jnp.float32)
    hs2_ref[...] = h2 * dinv[:, None]


_tc2 = pl.pallas_call(
    _tc2_body,
    grid=(_GRID,),
    in_specs=[
        pl.BlockSpec((_R, 128), lambda i: (i, 0)),
        pl.BlockSpec((2, _R, 128), lambda i: (0, i, 0)),
        pl.BlockSpec((2, _R, _DEGW), lambda i: (0, i, 0)),
        pl.BlockSpec((1, 128), lambda i: (0, 0)),
        pl.BlockSpec((128, 128), lambda i: (0, 0)),
    ],
    out_specs=pl.BlockSpec((_R, 128), lambda i: (i, 0)),
    out_shape=jax.ShapeDtypeStruct((_NPAD, 128), jnp.float32),
)


def _tc3_body(hs2_ref, p_ref, degp_ref, b2_ref, batch_ref, wl_ref, bl_ref,
              out_ref, sums, cnts):
    i = pl.program_id(0)
    dinv = _dinv_block(degp_ref[...])
    agg = (p_ref[0] + p_ref[1] + hs2_ref[...]) * dinv[:, None]
    out2 = jnp.maximum(agg + b2_ref[...], 0.0)

    seg = lax.broadcasted_iota(jnp.int32, (_G, _R), 0).astype(jnp.float32)
    m = (seg == batch_ref[...]).astype(jnp.float32)

    @pl.when(i == 0)
    def _():
        sums[...] = jnp.zeros_like(sums)
        cnts[...] = jnp.zeros_like(cnts)

    sums[...] += jnp.dot(m, out2, preferred_element_type=jnp.float32)
    cnts[...] += jnp.sum(m, axis=1, keepdims=True)

    @pl.when(i == _GRID - 1)
    def _():
        pooled = sums[...] / jnp.maximum(cnts[...][:, :1], 1.0)
        out_ref[...] = (jnp.dot(pooled, wl_ref[...],
                                preferred_element_type=jnp.float32)
                        + bl_ref[...])


_tc3 = pl.pallas_call(
    _tc3_body,
    grid=(_GRID,),
    in_specs=[
        pl.BlockSpec((_R, 128), lambda i: (i, 0)),
        pl.BlockSpec((2, _R, 128), lambda i: (0, i, 0)),
        pl.BlockSpec((2, _R, _DEGW), lambda i: (0, i, 0)),
        pl.BlockSpec((1, 128), lambda i: (0, 0)),
        pl.BlockSpec((1, _R), lambda i: (0, i)),
        pl.BlockSpec((128, _G), lambda i: (0, 0)),
        pl.BlockSpec((1, _G), lambda i: (0, 0)),
    ],
    out_specs=pl.BlockSpec((_G, _G), lambda i: (0, 0)),
    out_shape=jax.ShapeDtypeStruct((_G, _G), jnp.float32),
    scratch_shapes=[
        pltpu.VMEM((_G, 128), jnp.float32),
        pltpu.VMEM((_G, 128), jnp.float32),
    ],
)


def kernel(x, edge_index, batch, W1, b1, W2, b2, Wl, bl):
    f32 = jnp.float32
    xp = jnp.concatenate([x, jnp.zeros((_NPAD - _N, x.shape[1]), x.dtype)],
                         axis=0)
    pad_e = _EPAD - _E
    pad_iota = jnp.arange(pad_e, dtype=jnp.int32)
    src = jnp.concatenate(
        [edge_index[0], pad_iota % _N]).reshape(_NCH, _CHUNK)
    dst = jnp.concatenate(
        [edge_index[1], _N + pad_iota % (_NPAD - _N)]).reshape(_NCH, _CHUNK)
    batchp = jnp.concatenate(
        [batch, jnp.full((_NPAD - _N,), 255, batch.dtype)]
    ).astype(f32).reshape(1, _NPAD)

    w2p = jnp.zeros((128, 128), f32).at[:, :64].set(W2)
    b2p = jnp.zeros((1, 128), f32).at[0, :64].set(b2)
    wlp = jnp.zeros((128, _G), f32).at[:64, :].set(Wl)

    degp = _deg_kernel(dst)
    hs1 = _tc1(xp, W1, degp)
    p1 = _msg_kernel(hs1, src, dst)
    hs2 = _tc2(hs1, p1, degp, b1.reshape(1, -1), w2p)
    p2 = _msg_kernel(hs2, src, dst)
    out = _tc3(hs2, p2, degp, b2p, batchp, wlp, bl.reshape(1, -1))
    return out
